# baseline (speedup 1.0000x reference)
import functools

import jax
import jax.numpy as jnp
from jax import lax
from jax.experimental import pallas as pl
from jax.experimental.pallas import tpu as pltpu
from jax.experimental.pallas import tpu_sc as plsc

B = 4096
F = 26
V = 100000
D = 16
T = 8
P = F * (F - 1) // 2
PP = 384
NG = PP // 16
ROWS = B * F

NC = 2
NS = 16
NW = NC * NS
R_PER_W = ROWS // NW


def _sc_gather(flat_idx, fm_flat, lin_flat):
    mesh = plsc.VectorSubcoreMesh(core_axis_name="c", subcore_axis_name="s")

    @functools.partial(
        pl.kernel,
        mesh=mesh,
        compiler_params=pltpu.CompilerParams(use_tc_tiling_on_sc=False),
        out_type=[
            jax.ShapeDtypeStruct((ROWS, D), jnp.float32),
            jax.ShapeDtypeStruct((ROWS,), jnp.float32),
        ],
        scratch_types=[
            pltpu.VMEM((R_PER_W,), jnp.int32),
            pltpu.VMEM((R_PER_W, D), jnp.float32),
            pltpu.VMEM((R_PER_W,), jnp.float32),
            pltpu.SemaphoreType.DMA,
            pltpu.SemaphoreType.DMA,
        ],
    )
    def gather_k(idx_hbm, fm_hbm, lin_hbm, outf_hbm, outl_hbm,
                 idx_v, rows_v, lrows_v, sem_f, sem_l):
        wid = lax.axis_index("s") * NC + lax.axis_index("c")
        base = wid * R_PER_W
        pltpu.sync_copy(idx_hbm.at[pl.ds(base, R_PER_W)], idx_v)
        cp_f = pltpu.async_copy(fm_hbm.at[idx_v], rows_v, sem_f)
        cp_l = pltpu.async_copy(lin_hbm.at[idx_v], lrows_v, sem_l)
        cp_f.wait()
        cp_l.wait()
        pltpu.sync_copy(rows_v, outf_hbm.at[pl.ds(base, R_PER_W)])
        pltpu.sync_copy(lrows_v, outl_hbm.at[pl.ds(base, R_PER_W)])

    return gather_k(flat_idx, fm_flat, lin_flat)


def _dense_block(fm_ref, lin_ref, wq_ref, hbd_ref, btile_ref, pb_ref,
                 out_ref, p2_ref):
    bB = fm_ref.shape[0]
    e = fm_ref[...].astype(jnp.bfloat16)
    off = 0
    for g in range(1, F):
        w = (F - g) * D
        p2_ref[:, off:off + w] = e[:, :w] * e[:, g * D:g * D + w]
        off += w
    p2_ref[:, off:] = jnp.zeros((bB, PP * D - off), jnp.bfloat16)

    wq = wq_ref[...]
    hbd = hbd_ref[...]
    btile = btile_ref[...]
    sc_chunks = []
    q_chunks = []
    for gi in range(NG):
        pg = p2_ref[:, gi * 256:(gi + 1) * 256]
        mm = lax.dot_general(pg, wq, (((1,), (0,)), ((), ())),
                             preferred_element_type=jnp.float32)
        att = jnp.maximum(mm[:, :128] + btile, 0.0).astype(jnp.bfloat16)
        q_chunks.append(mm[:, 128:])
        sg = lax.dot_general(att, hbd, (((1,), (0,)), ((), ())),
                             preferred_element_type=jnp.float32)
        sc_chunks.append(sg)
    scores = jnp.concatenate(sc_chunks, axis=1)
    q = jnp.concatenate(q_chunks, axis=1)
    lane = lax.broadcasted_iota(jnp.int32, scores.shape, 1)
    scores = jnp.where(lane < P, scores, -1e30)
    m = jnp.max(scores, axis=1, keepdims=True)
    ex = jnp.exp(scores - m)
    s = jnp.sum(ex, axis=1, keepdims=True)
    afm = jnp.sum(ex * q, axis=1, keepdims=True) / s
    lin = jnp.sum(lin_ref[...], axis=1, keepdims=True)
    out_ref[...] = jax.nn.sigmoid(afm + lin + pb_ref[0, 0])


def _dense(fm2, lin2, wq, hbd, btile, pb, bB=512, interpret=False):
    grid = (B // bB,)
    return pl.pallas_call(
        _dense_block,
        grid=grid,
        in_specs=[
            pl.BlockSpec((bB, F * D), lambda i: (i, 0)),
            pl.BlockSpec((bB, F), lambda i: (i, 0)),
            pl.BlockSpec((256, 144), lambda i: (0, 0)),
            pl.BlockSpec((128, 16), lambda i: (0, 0)),
            pl.BlockSpec((1, 128), lambda i: (0, 0)),
            pl.BlockSpec(memory_space=pltpu.SMEM),
        ],
        out_specs=pl.BlockSpec((bB, 1), lambda i: (i, 0)),
        out_shape=jax.ShapeDtypeStruct((B, 1), jnp.float32),
        scratch_shapes=[pltpu.VMEM((bB, PP * D), jnp.bfloat16)],
        interpret=interpret,
    )(fm2, lin2, wq, hbd, btile, pb)


def _weights(att_W, att_b, att_h, proj_W, proj_b):
    eye = jnp.eye(16, dtype=jnp.float32)
    wbd = jnp.einsum('ij,dt->idjt', eye, att_W).reshape(256, 128)
    qbd = jnp.einsum('ij,d->idj', eye, proj_W[:, 0]).reshape(256, 16)
    wq = jnp.concatenate([wbd, qbd], axis=1).astype(jnp.bfloat16)
    hbd = jnp.einsum('ij,t->itj', eye, att_h).reshape(128, 16)
    hbd = hbd.astype(jnp.bfloat16)
    btile = jnp.tile(att_b, 16)[None, :].astype(jnp.float32)
    pb = proj_b.reshape(1, 1).astype(jnp.float32)
    return wq, hbd, btile, pb


def kernel(indices, fm_flat, lin_flat, att_W, att_b, att_h, proj_W, proj_b):
    offsets = jnp.arange(F, dtype=indices.dtype) * (V + 1)
    flat_idx = (indices + offsets[None, :]).reshape(ROWS).astype(jnp.int32)
    fm_rows, lin_rows = _sc_gather(flat_idx, fm_flat, lin_flat.reshape(-1))
    fm2 = fm_rows.reshape(B, F * D)
    lin2 = lin_rows.reshape(B, F)
    wq, hbd, btile, pb = _weights(att_W, att_b, att_h, proj_W, proj_b)
    return _dense(fm2, lin2, wq, hbd, btile, pb)


# speedup vs baseline: 3.7185x; 3.7185x over previous
import functools

import jax
import jax.numpy as jnp
from jax import lax
from jax.experimental import pallas as pl
from jax.experimental.pallas import tpu as pltpu
from jax.experimental.pallas import tpu_sc as plsc

B = 4096
F = 26
V = 100000
D = 16
T = 8
P = F * (F - 1) // 2
PP = 384
NG = PP // 16
ROWS = B * F

NC = 2
NS = 16
NW = NC * NS
R_PER_W = ROWS // NW


def _sc_gather(flat_idx, fm_flat, lin_flat):
    mesh = plsc.VectorSubcoreMesh(core_axis_name="c", subcore_axis_name="s")

    @functools.partial(
        pl.kernel,
        mesh=mesh,
        compiler_params=pltpu.CompilerParams(use_tc_tiling_on_sc=False),
        out_type=[
            jax.ShapeDtypeStruct((ROWS, D), jnp.float32),
            jax.ShapeDtypeStruct((ROWS,), jnp.float32),
        ],
        scratch_types=[
            pltpu.VMEM((R_PER_W,), jnp.int32),
            pltpu.VMEM((R_PER_W, D), jnp.float32),
            pltpu.VMEM((R_PER_W,), jnp.float32),
            pltpu.SemaphoreType.DMA,
            pltpu.SemaphoreType.DMA,
        ],
    )
    def gather_k(idx_hbm, fm_hbm, lin_hbm, outf_hbm, outl_hbm,
                 idx_v, rows_v, lrows_v, sem_f, sem_l):
        wid = lax.axis_index("s") * NC + lax.axis_index("c")
        base = wid * R_PER_W
        pltpu.sync_copy(idx_hbm.at[pl.ds(base, R_PER_W)], idx_v)
        cp_f = pltpu.async_copy(fm_hbm.at[idx_v], rows_v, sem_f)
        cp_l = pltpu.async_copy(lin_hbm.at[idx_v], lrows_v, sem_l)
        cp_f.wait()
        cp_l.wait()
        pltpu.sync_copy(rows_v, outf_hbm.at[pl.ds(base, R_PER_W)])
        pltpu.sync_copy(lrows_v, outl_hbm.at[pl.ds(base, R_PER_W)])

    return gather_k(flat_idx, fm_flat, lin_flat)


def _dense_block(fm_ref, lin_ref, wq_ref, hbd_ref, btile_ref, pb_ref,
                 out_ref, p2_ref):
    bB = fm_ref.shape[0]
    e = fm_ref[...].astype(jnp.bfloat16)
    off = 0
    for g in range(1, F):
        w = (F - g) * D
        p2_ref[:, off:off + w] = e[:, :w] * e[:, g * D:g * D + w]
        off += w
    p2_ref[:, off:] = jnp.zeros((bB, PP * D - off), jnp.bfloat16)

    wq = wq_ref[...]
    hbd = hbd_ref[...]
    btile = btile_ref[...]
    sc_chunks = []
    q_chunks = []
    for gi in range(NG):
        pg = p2_ref[:, gi * 256:(gi + 1) * 256]
        mm = lax.dot_general(pg, wq, (((1,), (0,)), ((), ())),
                             preferred_element_type=jnp.float32)
        att = jnp.maximum(mm[:, :128] + btile, 0.0).astype(jnp.bfloat16)
        q_chunks.append(mm[:, 128:])
        sg = lax.dot_general(att, hbd, (((1,), (0,)), ((), ())),
                             preferred_element_type=jnp.float32)
        sc_chunks.append(sg)
    scores = jnp.concatenate(sc_chunks, axis=1)
    q = jnp.concatenate(q_chunks, axis=1)
    lane = lax.broadcasted_iota(jnp.int32, scores.shape, 1)
    scores = jnp.where(lane < P, scores, -1e30)
    m = jnp.max(scores, axis=1, keepdims=True)
    ex = jnp.exp(scores - m)
    s = jnp.sum(ex, axis=1, keepdims=True)
    afm = jnp.sum(ex * q, axis=1, keepdims=True) / s
    lin = jnp.sum(lin_ref[...], axis=1, keepdims=True)
    out_ref[...] = jax.nn.sigmoid(afm + lin + pb_ref[0, 0])


def _dense(fm2, lin2, wq, hbd, btile, pb, bB=512, interpret=False):
    grid = (B // bB,)
    return pl.pallas_call(
        _dense_block,
        grid=grid,
        in_specs=[
            pl.BlockSpec((bB, F * D), lambda i: (i, 0)),
            pl.BlockSpec((bB, F), lambda i: (i, 0)),
            pl.BlockSpec((256, 144), lambda i: (0, 0)),
            pl.BlockSpec((128, 16), lambda i: (0, 0)),
            pl.BlockSpec((1, 128), lambda i: (0, 0)),
            pl.BlockSpec(memory_space=pltpu.SMEM),
        ],
        out_specs=pl.BlockSpec((bB, 1), lambda i: (i, 0)),
        out_shape=jax.ShapeDtypeStruct((B, 1), jnp.float32),
        scratch_shapes=[pltpu.VMEM((bB, PP * D), jnp.bfloat16)],
        interpret=interpret,
    )(fm2, lin2, wq, hbd, btile, pb)


def _weights(att_W, att_b, att_h, proj_W, proj_b):
    eye = jnp.eye(16, dtype=jnp.float32)
    wbd = jnp.einsum('ij,dt->idjt', eye, att_W).reshape(256, 128)
    qbd = jnp.einsum('ij,d->idj', eye, proj_W[:, 0]).reshape(256, 16)
    wq = jnp.concatenate([wbd, qbd], axis=1).astype(jnp.bfloat16)
    hbd = jnp.einsum('ij,t->itj', eye, att_h).reshape(128, 16)
    hbd = hbd.astype(jnp.bfloat16)
    btile = jnp.tile(att_b, 16)[None, :].astype(jnp.float32)
    pb = proj_b.reshape(1, 1).astype(jnp.float32)
    return wq, hbd, btile, pb


def kernel(indices, fm_flat, lin_flat, att_W, att_b, att_h, proj_W, proj_b):
    offsets = jnp.arange(F, dtype=indices.dtype) * (V + 1)
    flat_idx = (indices + offsets[None, :]).reshape(ROWS).astype(jnp.int32)
    fm_rows = jnp.take(fm_flat, flat_idx, axis=0)
    lin_rows = jnp.take(lin_flat.reshape(-1), flat_idx, axis=0)
    fm2 = fm_rows.reshape(B, F * D)
    lin2 = lin_rows.reshape(B, F)
    wq, hbd, btile, pb = _weights(att_W, att_b, att_h, proj_W, proj_b)
    return _dense(fm2, lin2, wq, hbd, btile, pb)
